# baseline (speedup 1.0000x reference)
import functools

import jax
import jax.numpy as jnp
from jax.experimental import pallas as pl
from jax.experimental.pallas import tpu as pltpu

_VMEM_LIMIT_BYTES = 48 * 1024 * 1024


def _attn_scale_block(x_ref, w1_ref, w2_ref, o_ref, *, inv_hw):
    x = x_ref[0]
    hw = x.shape[1]
    ones_col = jnp.ones((hw, 1), dtype=jnp.float32)
    s = jax.lax.dot(x, ones_col, preferred_element_type=jnp.float32)
    mx = jnp.max(x, axis=1, keepdims=True)
    pooled = jnp.concatenate([s * inv_hw, mx], axis=1)
    h = jnp.dot(w1_ref[...], pooled,
                preferred_element_type=jnp.float32)
    h = jnp.maximum(h, 0.0)
    z = jnp.dot(w2_ref[...], h,
                preferred_element_type=jnp.float32)
    att = jax.nn.sigmoid(z[:, 0:1] + z[:, 1:2])
    o_ref[0] = x * att


def kernel(x_nchw, w1, w2):
    N, C, H, W = x_nchw.shape
    HW = H * W
    Cr = w1.shape[0]
    x_k = x_nchw.reshape(N, C, HW)
    itemsize = jnp.dtype(x_k.dtype).itemsize
    cost = pl.CostEstimate(
        flops=2 * N * C * HW + N * (2 * C * HW) + 8 * N * C * Cr,
        transcendentals=N * C,
        bytes_accessed=2 * N * C * HW * itemsize + 2 * C * Cr * 4,
    )
    body = functools.partial(_attn_scale_block, inv_hw=1.0 / HW)
    out = pl.pallas_call(
        body,
        out_shape=jax.ShapeDtypeStruct((N, C, HW), x_k.dtype),
        grid=(N,),
        in_specs=[
            pl.BlockSpec((1, C, HW), lambda n: (n, 0, 0)),
            pl.BlockSpec((Cr, C), lambda n: (0, 0)),
            pl.BlockSpec((C, Cr), lambda n: (0, 0)),
        ],
        out_specs=pl.BlockSpec((1, C, HW), lambda n: (n, 0, 0)),
        compiler_params=pltpu.CompilerParams(
            dimension_semantics=("parallel",),
            vmem_limit_bytes=_VMEM_LIMIT_BYTES,
        ),
        cost_estimate=cost,
    )(x_k, w1, w2)
    return out.reshape(N, C, H, W)
